# Initial kernel scaffold; baseline (speedup 1.0000x reference)
import jax
import jax.numpy as jnp
from jax.experimental import pallas as pl


def kernel(token_ids, row_splits):
    raise NotImplementedError("write your pallas kernel here")


# speedup vs baseline: 9.1686x; 9.1686x over previous
import jax
import jax.numpy as jnp
from jax import lax
from jax.experimental import pallas as pl
from jax.experimental.pallas import tpu as pltpu
from jax.experimental.pallas import tpu_sc as plsc

B = 16
T = 16384
MAXSEQ = 2048
L = MAXSEQ + 2
L_PAD = 2176
CLS_ID = 2
SEP_ID = 3
LANES = 16
NCHUNK = (MAXSEQ + LANES) // LANES
WIN = 2080
PAD_T = 18464


def _row_body(tokens_hbm, meta_hbm, out_hbm, meta_v, win_v, row_v):
    c = lax.axis_index("c")
    s = lax.axis_index("s")
    wid = s * 2 + c

    @pl.when(wid < B)
    def _():
        pltpu.sync_copy(meta_hbm, meta_v.at[pl.ds(0, 2 * B)])
        lane = lax.broadcasted_iota(jnp.int32, (LANES,), 0)
        start = meta_v[pl.ds(wid, LANES)][0]
        ln = meta_v[pl.ds(wid + B, LANES)][0]
        start_al = (start // 8) * 8
        off = start - start_al
        pltpu.sync_copy(tokens_hbm.at[pl.ds(start_al, WIN)], win_v)

        def chunk(j, carry):
            idx = lane + j * LANES
            tok = win_v[pl.ds(off + j * LANES, LANES)]
            val = jnp.where(idx < ln, tok,
                            jnp.where(idx == ln,
                                      jnp.full((LANES,), SEP_ID, jnp.int32),
                                      jnp.zeros((LANES,), jnp.int32)))
            row_v[pl.ds(1 + j * LANES, LANES)] = val
            return carry

        lax.fori_loop(0, NCHUNK, chunk, 0)
        head = row_v[pl.ds(0, LANES)]
        row_v[pl.ds(0, LANES)] = jnp.where(
            lane == 0, jnp.full((LANES,), CLS_ID, jnp.int32), head)
        pltpu.sync_copy(row_v.at[pl.ds(0, L_PAD)], out_hbm.at[wid])


def kernel(token_ids, row_splits):
    tokens_pad = jnp.zeros((PAD_T,), jnp.int32).at[:T].set(token_ids)
    starts = row_splits[:B]
    lens = row_splits[1:B + 1] - starts
    meta = jnp.concatenate([starts, lens])
    mesh = plsc.VectorSubcoreMesh(core_axis_name="c", subcore_axis_name="s")
    f = pl.kernel(
        _row_body,
        out_type=jax.ShapeDtypeStruct((B, L_PAD), jnp.int32),
        mesh=mesh,
        scratch_types=[
            pltpu.VMEM((3 * B,), jnp.int32),
            pltpu.VMEM((WIN,), jnp.int32),
            pltpu.VMEM((L_PAD,), jnp.int32),
        ],
    )
    return f(tokens_pad, meta)[:, :L]
